# Initial kernel scaffold; baseline (speedup 1.0000x reference)
import jax
import jax.numpy as jnp
from jax.experimental import pallas as pl


def kernel(x, edge_index, neg_edge_index, Wl1, Wr1, att1, b1, Wl2, Wr2, att2, b2, Wl3, Wr3, att3, b3, Wlin1, blin1, Wlin2, blin2, c1, c2):
    raise NotImplementedError("write your pallas kernel here")


# speedup vs baseline: 1.2043x; 1.2043x over previous
import functools

import jax
import jax.numpy as jnp
from jax.experimental import pallas as pl


def _mm_body(x_ref, w_ref, o_ref):
    o_ref[...] = jnp.dot(x_ref[...], w_ref[...],
                         preferred_element_type=jnp.float32)


def _mm(x, w):
    return pl.pallas_call(
        _mm_body,
        out_shape=jax.ShapeDtypeStruct((x.shape[0], w.shape[1]), jnp.float32),
    )(x, w)


def _leaky(x):
    return jnp.where(x >= 0, x, 0.2 * x)


def _gatv2(x, ei, Wl, Wr, att, b):
    n = x.shape[0]
    loop = jnp.arange(n, dtype=ei.dtype)
    src = jnp.concatenate([ei[0], loop])
    dst = jnp.concatenate([ei[1], loop])
    xl = _mm(x, Wl)
    xr = _mm(x, Wr)
    e = jnp.sum(_leaky(xl[src] + xr[dst]) * att, axis=-1)
    m = jax.ops.segment_max(e, dst, num_segments=n)
    ex = jnp.exp(e - m[dst])
    s = jax.ops.segment_sum(ex, dst, num_segments=n)
    alpha = ex / (s[dst] + 1e-16)
    out = jax.ops.segment_sum(xl[src] * alpha[:, None], dst, num_segments=n)
    return out + b


def kernel(x, edge_index, neg_edge_index, Wl1, Wr1, att1, b1, Wl2, Wr2, att2, b2, Wl3, Wr3, att3, b3, Wlin1, blin1, Wlin2, blin2, c1, c2):
    x0 = x
    h = jax.nn.relu(_gatv2(x0, edge_index, Wl1, Wr1, att1, b1))
    x1 = jax.nn.relu(_gatv2(h, edge_index, Wl2, Wr2, att2, b2))
    xs = x1 + jax.nn.relu(_mm(x0, Wlin1) + blin1)
    z = x1 + jax.nn.relu(_mm(x0, Wlin2) + blin2)
    pos = jax.nn.sigmoid(jnp.sum(z[edge_index[0]] * z[edge_index[1]], axis=1))
    pos_loss = -jnp.mean(jnp.log(pos + 1e-15))
    neg = jax.nn.sigmoid(jnp.sum(z[neg_edge_index[0]] * z[neg_edge_index[1]], axis=1))
    neg_loss = -jnp.mean(jnp.log(1.0 - neg + 1e-15))
    r_loss = pos_loss + neg_loss
    out = _gatv2(xs, edge_index, Wl3, Wr3, att3, b3)
    return (out, r_loss, c1, c2)
